# baseline (speedup 1.0000x reference)
import jax
import jax.numpy as jnp
from jax import lax
from jax.experimental import pallas as pl
from jax.experimental.pallas import tpu as pltpu
from jax.experimental.pallas import tpu_sc as plsc

_LEVEL = 15
_DIM = 2
_BASE_RES = 512
_N = 524288
_OUT_COLS = 1 + _LEVEL * _DIM

_OFFSETS = []
_SCALES = []
_off = 0
for _i in range(_LEVEL):
    _res = int(_BASE_RES * 2.0 ** _i)
    _OFFSETS.append(_off)
    _SCALES.append(float(_res))
    _off += _res + 8
_TOTAL_ROWS = _off

_NC, _NS = 2, 16
_NW = _NC * _NS
_PTS_PER_TILE = _N // _NW
_C = 1024
_CHUNKS = _PTS_PER_TILE // _C
_G = _C // 16
_IDX_N = 2 * _C
_DMA_IDX = 128
_NDMA = _IDX_N // _DMA_IDX

_TS_LEVELS = 6
_TS_ROWS = _OFFSETS[_TS_LEVELS]
_SP_LEVELS = 2
_SP_BASE = _TS_ROWS
_SP_ROWS = _OFFSETS[_TS_LEVELS + _SP_LEVELS] - _SP_BASE
_DMA_LEVELS = list(range(_TS_LEVELS, _LEVEL))


def _mrl_body(
    x_hbm, d_hbm, out_hbm,
    x_v, idx_a, idx_b, vals_a, vals_b, out_v, st0_v, st1_v, sp_v,
    sem_a, sem_b, sem_s,
):
    cid = lax.axis_index("c")
    sid = lax.axis_index("s")
    wid = cid * _NS + sid
    iota16 = lax.iota(jnp.int32, 16)

    ts_copies = [
        pltpu.async_copy(d_hbm.at[pl.ds(0, _TS_ROWS)], st0_v, sem_s),
        pltpu.async_copy(d_hbm.at[pl.ds(_TOTAL_ROWS, _TS_ROWS)], st1_v, sem_s),
    ]
    for cp in ts_copies:
        cp.wait()

    @pl.when(sid == 0)
    def _stage_spmem():
        sp_copies = [
            pltpu.async_copy(
                d_hbm.at[pl.ds(_SP_BASE, _SP_ROWS)],
                sp_v.at[pl.ds(0, _SP_ROWS)], sem_s),
            pltpu.async_copy(
                d_hbm.at[pl.ds(_TOTAL_ROWS + _SP_BASE, _SP_ROWS)],
                sp_v.at[pl.ds(_SP_ROWS, _SP_ROWS)], sem_s),
        ]
        for cp in sp_copies:
            cp.wait()

    plsc.subcore_barrier()

    vals = {0: vals_a, 1: vals_b}
    idxs = {0: idx_a, 1: idx_b}
    sems = {0: sem_a, 1: sem_b}

    def chunk_body(ci, carry):
        base = (wid * _CHUNKS + ci) * _C
        pltpu.sync_copy(x_hbm.at[pl.ds(base, _C)], x_v)

        def build_and_fire(l):
            scale = _SCALES[l]
            spmem = l < _TS_LEVELS + _SP_LEVELS
            offl = _OFFSETS[l] - (_SP_BASE if spmem else 0)
            idx_v = idxs[l % 2]

            d1shift = _SP_ROWS if spmem else _TOTAL_ROWS

            def idx_body(g, c, scale=scale, offl=offl, idx_v=idx_v,
                         d1shift=d1shift):
                xv = x_v[pl.ds(g * 16, 16)]
                pos = jnp.minimum(jnp.maximum(xv, 0.0), 1.0) * scale
                i0 = pos.astype(jnp.int32) + offl
                idx_v[pl.ds(g * 16, 16)] = i0
                idx_v[pl.ds(_C + g * 16, 16)] = i0 + 1
                idx_v[pl.ds(2 * _C + g * 16, 16)] = i0 + d1shift
                idx_v[pl.ds(3 * _C + g * 16, 16)] = i0 + (d1shift + 1)
                return c

            lax.fori_loop(0, _G, idx_body, 0)

            src = sp_v if spmem else d_hbm
            buf, sem = vals[l % 2], sems[l % 2]
            copies = []
            for j in range(2 * _NDMA):
                isl = idx_v.at[pl.ds(j * _DMA_IDX, _DMA_IDX)]
                copies.append(
                    pltpu.async_copy(
                        src.at[isl],
                        buf.at[pl.ds(j * _DMA_IDX, _DMA_IDX)],
                        sem,
                    )
                )
            return copies

        def comp_dma_level(l):
            scale = _SCALES[l]
            buf = vals[l % 2]

            def comp_body(g, c, scale=scale, l=l, buf=buf):
                xv = x_v[pl.ds(g * 16, 16)]
                pos = jnp.minimum(jnp.maximum(xv, 0.0), 1.0) * scale
                i0 = pos.astype(jnp.int32)
                w1 = pos - i0.astype(jnp.float32)
                w0 = 1.0 - w1
                s = g * 16
                v0d0 = buf[pl.ds(s, 16)]
                v1d0 = buf[pl.ds(_C + s, 16)]
                v0d1 = buf[pl.ds(_IDX_N + s, 16)]
                v1d1 = buf[pl.ds(_IDX_N + _C + s, 16)]
                o0 = w0 * v0d0 + w1 * v1d0
                o1 = w0 * v0d1 + w1 * v1d1
                flat = (s + iota16) * _OUT_COLS + (1 + _DIM * l)
                plsc.store_scatter(out_v, [flat], o0)
                plsc.store_scatter(out_v, [flat + 1], o1)
                return c

            lax.fori_loop(0, _G, comp_body, 0)

        inflight = build_and_fire(_DMA_LEVELS[0])

        def xcol_body(g, c):
            xv = x_v[pl.ds(g * 16, 16)]
            flat = (g * 16 + iota16) * _OUT_COLS
            plsc.store_scatter(out_v, [flat], xv)
            return c

        lax.fori_loop(0, _G, xcol_body, 0)

        for l in range(_TS_LEVELS):
            scale = _SCALES[l]
            offl = _OFFSETS[l]

            def comp_staged(g, c, scale=scale, offl=offl, l=l):
                xv = x_v[pl.ds(g * 16, 16)]
                pos = jnp.minimum(jnp.maximum(xv, 0.0), 1.0) * scale
                i0 = pos.astype(jnp.int32)
                w1 = pos - i0.astype(jnp.float32)
                w0 = 1.0 - w1
                r0 = i0 + offl
                r1 = r0 + 1
                v0d0 = plsc.load_gather(st0_v, [r0])
                v1d0 = plsc.load_gather(st0_v, [r1])
                v0d1 = plsc.load_gather(st1_v, [r0])
                v1d1 = plsc.load_gather(st1_v, [r1])
                o0 = w0 * v0d0 + w1 * v1d0
                o1 = w0 * v0d1 + w1 * v1d1
                flat = (g * 16 + iota16) * _OUT_COLS + (1 + _DIM * l)
                plsc.store_scatter(out_v, [flat], o0)
                plsc.store_scatter(out_v, [flat + 1], o1)
                return c

            lax.fori_loop(0, _G, comp_staged, 0)

        for l in _DMA_LEVELS[1:]:
            nxt = build_and_fire(l)
            for cp in inflight:
                cp.wait()
            inflight = nxt
            comp_dma_level(l - 1)
        for cp in inflight:
            cp.wait()
        comp_dma_level(_DMA_LEVELS[-1])

        pltpu.sync_copy(out_v, out_hbm.at[pl.ds(base * _OUT_COLS, _C * _OUT_COLS)])
        return carry

    lax.fori_loop(0, _CHUNKS, chunk_body, 0)


_mrl_call = pl.kernel(
    _mrl_body,
    out_type=jax.ShapeDtypeStruct((_N * _OUT_COLS,), jnp.float32),
    mesh=plsc.VectorSubcoreMesh(core_axis_name="c", subcore_axis_name="s"),
    compiler_params=pltpu.CompilerParams(
        needs_layout_passes=False, use_tc_tiling_on_sc=False
    ),
    scratch_types=[
        pltpu.VMEM((_C,), jnp.float32),
        pltpu.VMEM((2 * _IDX_N,), jnp.int32),
        pltpu.VMEM((2 * _IDX_N,), jnp.int32),
        pltpu.VMEM((2 * _IDX_N,), jnp.float32),
        pltpu.VMEM((2 * _IDX_N,), jnp.float32),
        pltpu.VMEM((_C * _OUT_COLS,), jnp.float32),
        pltpu.VMEM((_TS_ROWS,), jnp.float32),
        pltpu.VMEM((_TS_ROWS,), jnp.float32),
        pltpu.VMEM_SHARED((2 * _SP_ROWS,), jnp.float32),
        pltpu.SemaphoreType.DMA,
        pltpu.SemaphoreType.DMA,
        pltpu.SemaphoreType.DMA,
    ],
)


def kernel(x, data):
    dt = data.T
    tbl = jnp.concatenate([dt[0], dt[1]])
    out = _mrl_call(x.reshape(-1), tbl)
    return out.reshape(_N, _OUT_COLS)


# speedup vs baseline: 1.0574x; 1.0574x over previous
import jax
import jax.numpy as jnp
from jax import lax
from jax.experimental import pallas as pl
from jax.experimental.pallas import tpu as pltpu
from jax.experimental.pallas import tpu_sc as plsc

_LEVEL = 15
_DIM = 2
_BASE_RES = 512
_N = 524288
_OUT_COLS = 1 + _LEVEL * _DIM

_OFFSETS = []
_SCALES = []
_off = 0
for _i in range(_LEVEL):
    _res = int(_BASE_RES * 2.0 ** _i)
    _OFFSETS.append(_off)
    _SCALES.append(float(_res))
    _off += _res + 8
_TOTAL_ROWS = _off

_NC, _NS = 2, 16
_NW = _NC * _NS
_PTS_PER_TILE = _N // _NW
_C = 1024
_CHUNKS = _PTS_PER_TILE // _C
_G = _C // 16
_IDX_N = 2 * _C
_DMA_IDX = 128
_NDMA = _IDX_N // _DMA_IDX

_TS_LEVELS = 6
_TS_ROWS = _OFFSETS[_TS_LEVELS]
_SP_LEVELS = 2
_SP_BASE = _TS_ROWS
_SP_ROWS = _OFFSETS[_TS_LEVELS + _SP_LEVELS] - _SP_BASE
_DMA_LEVELS = list(range(_TS_LEVELS, _LEVEL))


def _mrl_body(
    x_hbm, d0_hbm, d1_hbm, out_hbm,
    x_v, idx_a, idx_b, vals_a, vals_b, out_v, st0_v, st1_v, sp0_v, sp1_v,
    sem_a, sem_b, sem_s,
):
    cid = lax.axis_index("c")
    sid = lax.axis_index("s")
    wid = cid * _NS + sid
    iota16 = lax.iota(jnp.int32, 16)

    ts_copies = [
        pltpu.async_copy(d0_hbm.at[pl.ds(0, _TS_ROWS)], st0_v, sem_s),
        pltpu.async_copy(d1_hbm.at[pl.ds(0, _TS_ROWS)], st1_v, sem_s),
    ]
    for cp in ts_copies:
        cp.wait()

    @pl.when(sid == 0)
    def _stage_spmem():
        sp_copies = [
            pltpu.async_copy(d0_hbm.at[pl.ds(_SP_BASE, _SP_ROWS)], sp0_v, sem_s),
            pltpu.async_copy(d1_hbm.at[pl.ds(_SP_BASE, _SP_ROWS)], sp1_v, sem_s),
        ]
        for cp in sp_copies:
            cp.wait()

    plsc.subcore_barrier()

    vals = {0: vals_a, 1: vals_b}
    idxs = {0: idx_a, 1: idx_b}
    sems = {0: sem_a, 1: sem_b}

    def chunk_body(ci, carry):
        base = (wid * _CHUNKS + ci) * _C
        pltpu.sync_copy(x_hbm.at[pl.ds(base, _C)], x_v)

        def build_and_fire(l):
            scale = _SCALES[l]
            spmem = l < _TS_LEVELS + _SP_LEVELS
            offl = _OFFSETS[l] - (_SP_BASE if spmem else 0)
            idx_v = idxs[l % 2]

            def idx_body(g, c, scale=scale, offl=offl, idx_v=idx_v):
                xv = x_v[pl.ds(g * 16, 16)]
                pos = jnp.minimum(jnp.maximum(xv, 0.0), 1.0) * scale
                i0 = pos.astype(jnp.int32) + offl
                idx_v[pl.ds(g * 16, 16)] = i0
                idx_v[pl.ds(_C + g * 16, 16)] = i0 + 1
                return c

            lax.fori_loop(0, _G, idx_body, 0)

            s0 = sp0_v if spmem else d0_hbm
            s1 = sp1_v if spmem else d1_hbm
            buf, sem = vals[l % 2], sems[l % 2]
            copies = []
            for j in range(_NDMA):
                isl = idx_v.at[pl.ds(j * _DMA_IDX, _DMA_IDX)]
                copies.append(
                    pltpu.async_copy(
                        s0.at[isl],
                        buf.at[pl.ds(j * _DMA_IDX, _DMA_IDX)],
                        sem,
                    )
                )
                copies.append(
                    pltpu.async_copy(
                        s1.at[isl],
                        buf.at[pl.ds(_IDX_N + j * _DMA_IDX, _DMA_IDX)],
                        sem,
                    )
                )
            return copies

        def comp_dma_level(l):
            scale = _SCALES[l]
            buf = vals[l % 2]

            def comp_body(g, c, scale=scale, l=l, buf=buf):
                xv = x_v[pl.ds(g * 16, 16)]
                pos = jnp.minimum(jnp.maximum(xv, 0.0), 1.0) * scale
                i0 = pos.astype(jnp.int32)
                w1 = pos - i0.astype(jnp.float32)
                w0 = 1.0 - w1
                s = g * 16
                v0d0 = buf[pl.ds(s, 16)]
                v1d0 = buf[pl.ds(_C + s, 16)]
                v0d1 = buf[pl.ds(_IDX_N + s, 16)]
                v1d1 = buf[pl.ds(_IDX_N + _C + s, 16)]
                o0 = w0 * v0d0 + w1 * v1d0
                o1 = w0 * v0d1 + w1 * v1d1
                flat = (s + iota16) * _OUT_COLS + (1 + _DIM * l)
                plsc.store_scatter(out_v, [flat], o0)
                plsc.store_scatter(out_v, [flat + 1], o1)
                return c

            lax.fori_loop(0, _G, comp_body, 0)

        inflight = build_and_fire(_DMA_LEVELS[0])

        def xcol_body(g, c):
            xv = x_v[pl.ds(g * 16, 16)]
            flat = (g * 16 + iota16) * _OUT_COLS
            plsc.store_scatter(out_v, [flat], xv)
            return c

        lax.fori_loop(0, _G, xcol_body, 0)

        for l in range(_TS_LEVELS):
            scale = _SCALES[l]
            offl = _OFFSETS[l]

            def comp_staged(g, c, scale=scale, offl=offl, l=l):
                xv = x_v[pl.ds(g * 16, 16)]
                pos = jnp.minimum(jnp.maximum(xv, 0.0), 1.0) * scale
                i0 = pos.astype(jnp.int32)
                w1 = pos - i0.astype(jnp.float32)
                w0 = 1.0 - w1
                r0 = i0 + offl
                r1 = r0 + 1
                v0d0 = plsc.load_gather(st0_v, [r0])
                v1d0 = plsc.load_gather(st0_v, [r1])
                v0d1 = plsc.load_gather(st1_v, [r0])
                v1d1 = plsc.load_gather(st1_v, [r1])
                o0 = w0 * v0d0 + w1 * v1d0
                o1 = w0 * v0d1 + w1 * v1d1
                flat = (g * 16 + iota16) * _OUT_COLS + (1 + _DIM * l)
                plsc.store_scatter(out_v, [flat], o0)
                plsc.store_scatter(out_v, [flat + 1], o1)
                return c

            lax.fori_loop(0, _G, comp_staged, 0)

        for l in _DMA_LEVELS[1:]:
            nxt = build_and_fire(l)
            for cp in inflight:
                cp.wait()
            inflight = nxt
            comp_dma_level(l - 1)
        for cp in inflight:
            cp.wait()
        comp_dma_level(_DMA_LEVELS[-1])

        pltpu.sync_copy(out_v, out_hbm.at[pl.ds(base * _OUT_COLS, _C * _OUT_COLS)])
        return carry

    lax.fori_loop(0, _CHUNKS, chunk_body, 0)


_mrl_call = pl.kernel(
    _mrl_body,
    out_type=jax.ShapeDtypeStruct((_N * _OUT_COLS,), jnp.float32),
    mesh=plsc.VectorSubcoreMesh(core_axis_name="c", subcore_axis_name="s"),
    compiler_params=pltpu.CompilerParams(
        needs_layout_passes=False, use_tc_tiling_on_sc=False
    ),
    scratch_types=[
        pltpu.VMEM((_C,), jnp.float32),
        pltpu.VMEM((_IDX_N,), jnp.int32),
        pltpu.VMEM((_IDX_N,), jnp.int32),
        pltpu.VMEM((2 * _IDX_N,), jnp.float32),
        pltpu.VMEM((2 * _IDX_N,), jnp.float32),
        pltpu.VMEM((_C * _OUT_COLS,), jnp.float32),
        pltpu.VMEM((_TS_ROWS,), jnp.float32),
        pltpu.VMEM((_TS_ROWS,), jnp.float32),
        pltpu.VMEM_SHARED((_SP_ROWS,), jnp.float32),
        pltpu.VMEM_SHARED((_SP_ROWS,), jnp.float32),
        pltpu.SemaphoreType.DMA,
        pltpu.SemaphoreType.DMA,
        pltpu.SemaphoreType.DMA,
    ],
)


def kernel(x, data):
    dt = data.T
    out = _mrl_call(x.reshape(-1), dt[0], dt[1])
    return out.reshape(_N, _OUT_COLS)


# speedup vs baseline: 1.0698x; 1.0117x over previous
import jax
import jax.numpy as jnp
from jax import lax
from jax.experimental import pallas as pl
from jax.experimental.pallas import tpu as pltpu
from jax.experimental.pallas import tpu_sc as plsc

_LEVEL = 15
_DIM = 2
_BASE_RES = 512
_N = 524288
_OUT_COLS = 1 + _LEVEL * _DIM

_OFFSETS = []
_SCALES = []
_off = 0
for _i in range(_LEVEL):
    _res = int(_BASE_RES * 2.0 ** _i)
    _OFFSETS.append(_off)
    _SCALES.append(float(_res))
    _off += _res + 8
_TOTAL_ROWS = _off

_NC, _NS = 2, 16
_NW = _NC * _NS
_PTS_PER_TILE = _N // _NW
_C = 1024
_CHUNKS = _PTS_PER_TILE // _C
_G = _C // 16
_IDX_N = 2 * _C
_DMA_IDX = 128
_NDMA = _IDX_N // _DMA_IDX

_TS_LEVELS = 6
_TS_ROWS = _OFFSETS[_TS_LEVELS]
_SP_LEVELS = 2
_SP_BASE = _TS_ROWS
_SP_ROWS = _OFFSETS[_TS_LEVELS + _SP_LEVELS] - _SP_BASE
_DMA_LEVELS = list(range(_TS_LEVELS, _LEVEL))


def _mrl_body(
    x_hbm, d0_hbm, d1_hbm, out_hbm,
    x_v, idx_a, idx_b, vals_a, vals_b, out_v, st0_v, st1_v, sp0_v, sp1_v,
    sem_a, sem_b, sem_s, sem_o,
):
    cid = lax.axis_index("c")
    sid = lax.axis_index("s")
    wid = cid * _NS + sid
    iota16 = lax.iota(jnp.int32, 16)

    ts_copies = [
        pltpu.async_copy(d0_hbm.at[pl.ds(0, _TS_ROWS)], st0_v, sem_s),
        pltpu.async_copy(d1_hbm.at[pl.ds(0, _TS_ROWS)], st1_v, sem_s),
    ]
    for cp in ts_copies:
        cp.wait()

    @pl.when(sid == 0)
    def _stage_spmem():
        sp_copies = [
            pltpu.async_copy(d0_hbm.at[pl.ds(_SP_BASE, _SP_ROWS)], sp0_v, sem_s),
            pltpu.async_copy(d1_hbm.at[pl.ds(_SP_BASE, _SP_ROWS)], sp1_v, sem_s),
        ]
        for cp in sp_copies:
            cp.wait()

    plsc.subcore_barrier()

    vals = {0: vals_a, 1: vals_b}
    idxs = {0: idx_a, 1: idx_b}
    sems = {0: sem_a, 1: sem_b}

    def chunk_body(ci, carry):
        base = (wid * _CHUNKS + ci) * _C
        pltpu.sync_copy(x_hbm.at[pl.ds(base, _C)], x_v)

        def build_and_fire(l):
            scale = _SCALES[l]
            spmem = l < _TS_LEVELS + _SP_LEVELS
            offl = _OFFSETS[l] - (_SP_BASE if spmem else 0)
            idx_v = idxs[l % 2]

            def idx_body(g, c, scale=scale, offl=offl, idx_v=idx_v):
                xv = x_v[pl.ds(g * 16, 16)]
                pos = jnp.minimum(jnp.maximum(xv, 0.0), 1.0) * scale
                i0 = pos.astype(jnp.int32) + offl
                idx_v[pl.ds(g * 16, 16)] = i0
                idx_v[pl.ds(_C + g * 16, 16)] = i0 + 1
                return c

            lax.fori_loop(0, _G, idx_body, 0)

            s0 = sp0_v if spmem else d0_hbm
            s1 = sp1_v if spmem else d1_hbm
            buf, sem = vals[l % 2], sems[l % 2]
            copies = []
            for j in range(_NDMA):
                isl = idx_v.at[pl.ds(j * _DMA_IDX, _DMA_IDX)]
                copies.append(
                    pltpu.async_copy(
                        s0.at[isl],
                        buf.at[pl.ds(j * _DMA_IDX, _DMA_IDX)],
                        sem,
                    )
                )
                copies.append(
                    pltpu.async_copy(
                        s1.at[isl],
                        buf.at[pl.ds(_IDX_N + j * _DMA_IDX, _DMA_IDX)],
                        sem,
                    )
                )
            return copies

        def comp_dma_level(l):
            scale = _SCALES[l]
            buf = vals[l % 2]

            def comp_body(g, c, scale=scale, l=l, buf=buf):
                xv = x_v[pl.ds(g * 16, 16)]
                pos = jnp.minimum(jnp.maximum(xv, 0.0), 1.0) * scale
                i0 = pos.astype(jnp.int32)
                w1 = pos - i0.astype(jnp.float32)
                w0 = 1.0 - w1
                s = g * 16
                v0d0 = buf[pl.ds(s, 16)]
                v1d0 = buf[pl.ds(_C + s, 16)]
                v0d1 = buf[pl.ds(_IDX_N + s, 16)]
                v1d1 = buf[pl.ds(_IDX_N + _C + s, 16)]
                o0 = w0 * v0d0 + w1 * v1d0
                o1 = w0 * v0d1 + w1 * v1d1
                flat = (s + iota16) * _OUT_COLS + (1 + _DIM * l)
                plsc.store_scatter(out_v, [flat], o0)
                plsc.store_scatter(out_v, [flat + 1], o1)
                return c

            lax.fori_loop(0, _G, comp_body, 0)

        inflight = build_and_fire(_DMA_LEVELS[0])

        @pl.when(ci > 0)
        def _drain_out():
            pltpu.make_async_copy(
                out_v, out_hbm.at[pl.ds(base * _OUT_COLS, _C * _OUT_COLS)],
                sem_o).wait()

        def xcol_body(g, c):
            xv = x_v[pl.ds(g * 16, 16)]
            flat = (g * 16 + iota16) * _OUT_COLS
            plsc.store_scatter(out_v, [flat], xv)
            return c

        lax.fori_loop(0, _G, xcol_body, 0)

        for l in range(_TS_LEVELS):
            scale = _SCALES[l]
            offl = _OFFSETS[l]

            def comp_staged(g, c, scale=scale, offl=offl, l=l):
                xv = x_v[pl.ds(g * 16, 16)]
                pos = jnp.minimum(jnp.maximum(xv, 0.0), 1.0) * scale
                i0 = pos.astype(jnp.int32)
                w1 = pos - i0.astype(jnp.float32)
                w0 = 1.0 - w1
                r0 = i0 + offl
                r1 = r0 + 1
                v0d0 = plsc.load_gather(st0_v, [r0])
                v1d0 = plsc.load_gather(st0_v, [r1])
                v0d1 = plsc.load_gather(st1_v, [r0])
                v1d1 = plsc.load_gather(st1_v, [r1])
                o0 = w0 * v0d0 + w1 * v1d0
                o1 = w0 * v0d1 + w1 * v1d1
                flat = (g * 16 + iota16) * _OUT_COLS + (1 + _DIM * l)
                plsc.store_scatter(out_v, [flat], o0)
                plsc.store_scatter(out_v, [flat + 1], o1)
                return c

            lax.fori_loop(0, _G, comp_staged, 0)

        for l in _DMA_LEVELS[1:]:
            nxt = build_and_fire(l)
            for cp in inflight:
                cp.wait()
            inflight = nxt
            comp_dma_level(l - 1)
        for cp in inflight:
            cp.wait()
        comp_dma_level(_DMA_LEVELS[-1])

        pltpu.async_copy(
            out_v, out_hbm.at[pl.ds(base * _OUT_COLS, _C * _OUT_COLS)], sem_o)
        return carry

    lax.fori_loop(0, _CHUNKS, chunk_body, 0)

    last_base = (wid * _CHUNKS + _CHUNKS - 1) * _C
    pltpu.make_async_copy(
        out_v, out_hbm.at[pl.ds(last_base * _OUT_COLS, _C * _OUT_COLS)],
        sem_o).wait()


_mrl_call = pl.kernel(
    _mrl_body,
    out_type=jax.ShapeDtypeStruct((_N * _OUT_COLS,), jnp.float32),
    mesh=plsc.VectorSubcoreMesh(core_axis_name="c", subcore_axis_name="s"),
    compiler_params=pltpu.CompilerParams(
        needs_layout_passes=False, use_tc_tiling_on_sc=False
    ),
    scratch_types=[
        pltpu.VMEM((_C,), jnp.float32),
        pltpu.VMEM((_IDX_N,), jnp.int32),
        pltpu.VMEM((_IDX_N,), jnp.int32),
        pltpu.VMEM((2 * _IDX_N,), jnp.float32),
        pltpu.VMEM((2 * _IDX_N,), jnp.float32),
        pltpu.VMEM((_C * _OUT_COLS,), jnp.float32),
        pltpu.VMEM((_TS_ROWS,), jnp.float32),
        pltpu.VMEM((_TS_ROWS,), jnp.float32),
        pltpu.VMEM_SHARED((_SP_ROWS,), jnp.float32),
        pltpu.VMEM_SHARED((_SP_ROWS,), jnp.float32),
        pltpu.SemaphoreType.DMA,
        pltpu.SemaphoreType.DMA,
        pltpu.SemaphoreType.DMA,
        pltpu.SemaphoreType.DMA,
    ],
)


def kernel(x, data):
    dt = data.T
    out = _mrl_call(x.reshape(-1), dt[0], dt[1])
    return out.reshape(_N, _OUT_COLS)
